# baseline (speedup 1.0000x reference)
import jax
import jax.numpy as jnp
from jax import lax
from jax.experimental import pallas as pl
from jax.experimental.pallas import tpu as pltpu
from jax.experimental.pallas import tpu_sc as plsc

N = 10000
D = 128
B = 8
OUT = 128
NUM_MAT = 5
N_BLK = 2000

NC = 2
NS = 16
W = 128
CW = 10
CHE = CW * W
NCH = 16
NWIN = NCH * CW
EPS = NWIN * W
EPAD = EPS * NS
E = 320000
CHUNKS_PER_SC = B // NC
RS = 624
R_REM = N - NS * RS
NZ = 640

_GATHER_DNUMS = lax.GatherDimensionNumbers(
    offset_dims=(), collapsed_slice_dims=(0,), start_index_map=(0,))


def _bcast_lane(vec16, e):
    idx = jnp.full((16, 1), e, jnp.int32)
    return lax.gather(vec16, idx, _GATHER_DNUMS, slice_sizes=(1,),
                      mode=lax.GatherScatterMode.PROMISE_IN_BOUNDS)


def _spmm_body(x_hbm, src_hbm, dst_hbm, val_hbm, zeros_hbm, out_hbm,
               acc_sh, src_c, dst_c, val_c,
               r0, r1, r2, r3, sc0, sc1, dw0, dw1,
               g0, g1, g2, g3, ss0, ss1):
    cid = lax.axis_index("c")
    sid = lax.axis_index("s")
    ebase = sid * EPS
    rows = (r0, r1, r2, r3)
    gsem = (g0, g1, g2, g3)
    scaled = (sc0, sc1)
    ssem = (ss0, ss1)
    dw = (dw0, dw1)

    def emit_chunk(t, row_off):
        cbase = ebase + t * CHE
        pltpu.sync_copy(src_hbm.at[pl.ds(cbase, CHE)], src_c)
        pltpu.sync_copy(dst_hbm.at[pl.ds(cbase, CHE)], dst_c)
        pltpu.sync_copy(val_hbm.at[pl.ds(cbase, CHE)], val_c)

        @pl.loop(0, CHE // 16)
        def _(g):
            sl = pl.ds(g * 16, 16)
            src_c[sl] = src_c[sl] + row_off

        gd = [None] * CW
        for i in range(4):
            gd[i] = pltpu.async_copy(
                x_hbm.at[src_c.at[pl.ds(i * W, W)]], rows[i], gsem[i])

        for w in range(CW):
            p3, p2 = w % 4, w % 2
            gd[w].wait()

            def drain(p=p2):
                pltpu.make_async_copy(
                    scaled[p], acc_sh.at[dw[p]], ssem[p]).wait()

            if w >= 2:
                drain()
            else:
                pl.when(t > 0)(drain)

            @pl.loop(0, W // 16)
            def _(g):
                dw[p2][pl.ds(g * 16, 16)] = dst_c[pl.ds(w * W + g * 16, 16)]

            @pl.loop(0, W // 16)
            def _(g):
                v16 = val_c[pl.ds(w * W + g * 16, 16)]
                for e in range(16):
                    bc = plsc.pack(*[_bcast_lane(v16, e)] * 2,
                                   format=plsc.PackFormat.INTERLEAVED)
                    r = g * 16 + e
                    for q in range(2):
                        v64 = rows[p3][r, pl.ds(q * 64, 64)]
                        a, b = plsc.unpack(
                            v64, format=plsc.PackFormat.INTERLEAVED,
                            preferred_element_type=jnp.bfloat16)
                        scaled[p2][r, pl.ds(q * 64, 32)] = a * bc
                        scaled[p2][r, pl.ds(q * 64 + 32, 32)] = b * bc

            if w < CW - 4:
                gd[w + 4] = pltpu.async_copy(
                    x_hbm.at[src_c.at[pl.ds((w + 4) * W, W)]],
                    rows[p3], gsem[p3])

            pltpu.async_copy(scaled[p2], acc_sh.at[dw[p2]], ssem[p2],
                             add=True)

    @pl.loop(0, CHUNKS_PER_SC)
    def _(bi):
        bb = cid * CHUNKS_PER_SC + bi
        row_off = bb * N

        pltpu.sync_copy(zeros_hbm.at[pl.ds(0, RS)],
                        acc_sh.at[pl.ds(sid * RS, RS)])

        @pl.when(sid == 0)
        def _():
            pltpu.sync_copy(zeros_hbm.at[pl.ds(0, R_REM)],
                            acc_sh.at[pl.ds(NS * RS, R_REM)])
        plsc.subcore_barrier()

        @pl.loop(0, NCH)
        def _(t):
            emit_chunk(t, row_off)

        for p2 in range(2):
            pltpu.make_async_copy(
                scaled[p2], acc_sh.at[dw[p2]], ssem[p2]).wait()
        plsc.subcore_barrier()

        pltpu.sync_copy(acc_sh.at[pl.ds(sid * RS, RS)],
                        out_hbm.at[bb].at[pl.ds(sid * RS, RS)])

        @pl.when(sid == 0)
        def _():
            pltpu.sync_copy(acc_sh.at[pl.ds(NS * RS, R_REM)],
                            out_hbm.at[bb].at[pl.ds(NS * RS, R_REM)])
        plsc.subcore_barrier()


@jax.jit
def _spmm_sc(x_cm, src, dst, val):
    mesh = plsc.VectorSubcoreMesh(core_axis_name="c", subcore_axis_name="s")
    x_flat = x_cm.reshape(B * N, D)
    cp = pltpu.CompilerParams(needs_layout_passes=False,
                              use_tc_tiling_on_sc=False)
    kern = pl.kernel(
        _spmm_body,
        out_type=jax.ShapeDtypeStruct((B, N, D), jnp.bfloat16),
        mesh=mesh,
        compiler_params=cp,
        scratch_types=[
            pltpu.VMEM_SHARED((N, D), jnp.bfloat16),
            pltpu.VMEM((CHE,), jnp.int32),
            pltpu.VMEM((CHE,), jnp.int32),
            pltpu.VMEM((CHE,), jnp.float32),
            pltpu.VMEM((W, D), jnp.float8_e4m3fn),
            pltpu.VMEM((W, D), jnp.float8_e4m3fn),
            pltpu.VMEM((W, D), jnp.float8_e4m3fn),
            pltpu.VMEM((W, D), jnp.float8_e4m3fn),
            pltpu.VMEM((W, D), jnp.bfloat16),
            pltpu.VMEM((W, D), jnp.bfloat16),
            pltpu.VMEM((W,), jnp.int32),
            pltpu.VMEM((W,), jnp.int32),
            pltpu.SemaphoreType.DMA,
            pltpu.SemaphoreType.DMA,
            pltpu.SemaphoreType.DMA,
            pltpu.SemaphoreType.DMA,
            pltpu.SemaphoreType.DMA,
            pltpu.SemaphoreType.DMA,
        ],
    )
    zeros = jnp.zeros((NZ, D), jnp.bfloat16)
    return kern(x_flat, src, dst, val, zeros)


def _matmul_body(x0_ref, s11_ref, s21_ref, s12_ref, s22_ref, w0_ref,
                 wb_ref, b_ref, o_ref):
    acc = jnp.dot(x0_ref[0], w0_ref[...], preferred_element_type=jnp.float32)
    acc += jnp.dot(s11_ref[0], wb_ref[0], preferred_element_type=jnp.float32)
    acc += jnp.dot(s21_ref[0], wb_ref[1], preferred_element_type=jnp.float32)
    acc += jnp.dot(s12_ref[0], wb_ref[2], preferred_element_type=jnp.float32)
    acc += jnp.dot(s22_ref[0], wb_ref[3], preferred_element_type=jnp.float32)
    o_ref[0] = acc + b_ref[...]


def _final_matmul(x0, s11, s21, s12, s22, w0, wb, biases):
    f32_spec = pl.BlockSpec((1, N_BLK, D), lambda b, n: (b, n, 0))
    return pl.pallas_call(
        _matmul_body,
        grid=(B, N // N_BLK),
        in_specs=[f32_spec] * 5 + [
            pl.BlockSpec((D, OUT), lambda b, n: (0, 0)),
            pl.BlockSpec((4, D, OUT), lambda b, n: (0, 0, 0)),
            pl.BlockSpec((OUT,), lambda b, n: (0,)),
        ],
        out_specs=pl.BlockSpec((1, N_BLK, OUT), lambda b, n: (b, n, 0)),
        out_shape=jax.ShapeDtypeStruct((B, N, OUT), jnp.float32),
    )(x0, s11, s21, s12, s22, w0, wb, biases)


def _f8_table(arr):
    blk = jnp.stack([jnp.arange(32, dtype=jnp.int32),
                     32 + jnp.arange(32, dtype=jnp.int32)], axis=1).reshape(64)
    perm = jnp.concatenate([q * 64 + blk for q in range(D // 64)])
    return arr.astype(jnp.float8_e4m3fn)[..., perm]


def _pad_edges(src, dst, val):
    pad = EPAD - E
    fill = (jnp.arange(pad, dtype=jnp.int32) * 131) % N
    return (jnp.concatenate([src, fill]), jnp.concatenate([dst, fill]),
            jnp.pad(val, (0, pad)))


def kernel(x, edge_src1, edge_dst1, edge_val1, edge_src2, edge_dst2,
           edge_val2, weight, biases):
    b, n, d = x.shape
    s1, d1, v1 = _pad_edges(edge_src1, edge_dst1, edge_val1)
    s2, d2, v2 = _pad_edges(edge_src2, edge_dst2, edge_val2)

    x8 = _f8_table(x)
    s11 = _spmm_sc(x8, s1, d1, v1)
    s21 = _spmm_sc(_f8_table(s11), s1, d1, v1)
    s12 = _spmm_sc(x8, s2, d2, v2)
    s22 = _spmm_sc(_f8_table(s12), s2, d2, v2)

    w = weight.reshape(d, NUM_MAT, OUT)
    w0 = w[:, 0] - w[:, 2] - w[:, 4]
    wb = jnp.stack([
        w[:, 1],
        2.0 * w[:, 2],
        w[:, 3],
        2.0 * w[:, 4],
    ]).astype(jnp.bfloat16)
    return _final_matmul(x, s11, s21, s12, s22, w0, wb, biases)


# speedup vs baseline: 1.8251x; 1.8251x over previous
import jax
import jax.numpy as jnp
from jax import lax
from jax.experimental import pallas as pl
from jax.experimental.pallas import tpu as pltpu
from jax.experimental.pallas import tpu_sc as plsc

N = 10000
D = 128
B = 8
OUT = 128
NUM_MAT = 5
N_BLK = 2000

NC = 2
NS = 16
W = 128
CW = 10
CHE = CW * W
NCH = 16
NWIN = NCH * CW
EPS = NWIN * W
EPAD = EPS * NS
E = 320000
CHUNKS_PER_SC = B // NC
RS = 624
R_REM = N - NS * RS
NZ = 640

_GATHER_DNUMS = lax.GatherDimensionNumbers(
    offset_dims=(), collapsed_slice_dims=(0,), start_index_map=(0,))


def _bcast_lane(vec16, e):
    idx = jnp.full((16, 1), e, jnp.int32)
    return lax.gather(vec16, idx, _GATHER_DNUMS, slice_sizes=(1,),
                      mode=lax.GatherScatterMode.PROMISE_IN_BOUNDS)


def _spmm_body(x_hbm, src_hbm, dst_hbm, val_hbm, zeros_hbm, out_hbm,
               acc_sh, src_c, dst_c, val_c,
               r0, r1, r2, r3, sc0, sc1, dw0, dw1,
               g0, g1, g2, g3, ss0, ss1):
    cid = lax.axis_index("c")
    sid = lax.axis_index("s")
    ebase = sid * EPS
    rows = (r0, r1, r2, r3)
    gsem = (g0, g1, g2, g3)
    scaled = (sc0, sc1)
    ssem = (ss0, ss1)
    dw = (dw0, dw1)

    def emit_chunk(t, row_off):
        cbase = ebase + t * CHE
        pltpu.sync_copy(src_hbm.at[pl.ds(cbase, CHE)], src_c)
        pltpu.sync_copy(dst_hbm.at[pl.ds(cbase, CHE)], dst_c)
        pltpu.sync_copy(val_hbm.at[pl.ds(cbase, CHE)], val_c)

        @pl.loop(0, CHE // 16)
        def _(g):
            sl = pl.ds(g * 16, 16)
            src_c[sl] = src_c[sl] + row_off

        gd = [None] * CW
        for i in range(4):
            gd[i] = pltpu.async_copy(
                x_hbm.at[src_c.at[pl.ds(i * W, W)]], rows[i], gsem[i])

        for w in range(CW):
            p3, p2 = w % 4, w % 2
            gd[w].wait()

            def drain(p=p2):
                pltpu.make_async_copy(
                    scaled[p], acc_sh.at[dw[p]], ssem[p]).wait()

            if w >= 2:
                drain()
            else:
                pl.when(t > 0)(drain)

            @pl.loop(0, W // 16)
            def _(g):
                dw[p2][pl.ds(g * 16, 16)] = dst_c[pl.ds(w * W + g * 16, 16)]

            @pl.loop(0, W // 16)
            def _(g):
                v16 = val_c[pl.ds(w * W + g * 16, 16)]
                for e in range(16):
                    bc = plsc.pack(*[_bcast_lane(v16, e)] * 2,
                                   format=plsc.PackFormat.INTERLEAVED)
                    r = g * 16 + e
                    for q in range(4):
                        sl = pl.ds(q * 32, 32)
                        scaled[p2][r, sl] = rows[p3][r, sl] * bc

            if w < CW - 4:
                gd[w + 4] = pltpu.async_copy(
                    x_hbm.at[src_c.at[pl.ds((w + 4) * W, W)]],
                    rows[p3], gsem[p3])

            pltpu.async_copy(scaled[p2], acc_sh.at[dw[p2]], ssem[p2],
                             add=True)

    @pl.loop(0, CHUNKS_PER_SC)
    def _(bi):
        bb = cid * CHUNKS_PER_SC + bi
        row_off = bb * N

        pltpu.sync_copy(zeros_hbm.at[pl.ds(0, RS)],
                        acc_sh.at[pl.ds(sid * RS, RS)])

        @pl.when(sid == 0)
        def _():
            pltpu.sync_copy(zeros_hbm.at[pl.ds(0, R_REM)],
                            acc_sh.at[pl.ds(NS * RS, R_REM)])
        plsc.subcore_barrier()

        @pl.loop(0, NCH)
        def _(t):
            emit_chunk(t, row_off)

        for p2 in range(2):
            pltpu.make_async_copy(
                scaled[p2], acc_sh.at[dw[p2]], ssem[p2]).wait()
        plsc.subcore_barrier()

        pltpu.sync_copy(acc_sh.at[pl.ds(sid * RS, RS)],
                        out_hbm.at[bb].at[pl.ds(sid * RS, RS)])

        @pl.when(sid == 0)
        def _():
            pltpu.sync_copy(acc_sh.at[pl.ds(NS * RS, R_REM)],
                            out_hbm.at[bb].at[pl.ds(NS * RS, R_REM)])
        plsc.subcore_barrier()


@jax.jit
def _spmm_sc(x_cm, src, dst, val):
    mesh = plsc.VectorSubcoreMesh(core_axis_name="c", subcore_axis_name="s")
    x_flat = x_cm.reshape(B * N, D)
    cp = pltpu.CompilerParams(needs_layout_passes=False,
                              use_tc_tiling_on_sc=False)
    kern = pl.kernel(
        _spmm_body,
        out_type=jax.ShapeDtypeStruct((B, N, D), jnp.bfloat16),
        mesh=mesh,
        compiler_params=cp,
        scratch_types=[
            pltpu.VMEM_SHARED((N, D), jnp.bfloat16),
            pltpu.VMEM((CHE,), jnp.int32),
            pltpu.VMEM((CHE,), jnp.int32),
            pltpu.VMEM((CHE,), jnp.float32),
            pltpu.VMEM((W, D), jnp.bfloat16),
            pltpu.VMEM((W, D), jnp.bfloat16),
            pltpu.VMEM((W, D), jnp.bfloat16),
            pltpu.VMEM((W, D), jnp.bfloat16),
            pltpu.VMEM((W, D), jnp.bfloat16),
            pltpu.VMEM((W, D), jnp.bfloat16),
            pltpu.VMEM((W,), jnp.int32),
            pltpu.VMEM((W,), jnp.int32),
            pltpu.SemaphoreType.DMA,
            pltpu.SemaphoreType.DMA,
            pltpu.SemaphoreType.DMA,
            pltpu.SemaphoreType.DMA,
            pltpu.SemaphoreType.DMA,
            pltpu.SemaphoreType.DMA,
        ],
    )
    zeros = jnp.zeros((NZ, D), jnp.bfloat16)
    return kern(x_flat, src, dst, val, zeros)


def _matmul_body(x0_ref, s11_ref, s21_ref, s12_ref, s22_ref, w0_ref,
                 wb_ref, b_ref, o_ref):
    acc = jnp.dot(x0_ref[0], w0_ref[...], preferred_element_type=jnp.float32)
    acc += jnp.dot(s11_ref[0], wb_ref[0], preferred_element_type=jnp.float32)
    acc += jnp.dot(s21_ref[0], wb_ref[1], preferred_element_type=jnp.float32)
    acc += jnp.dot(s12_ref[0], wb_ref[2], preferred_element_type=jnp.float32)
    acc += jnp.dot(s22_ref[0], wb_ref[3], preferred_element_type=jnp.float32)
    o_ref[0] = acc + b_ref[...]


def _final_matmul(x0, s11, s21, s12, s22, w0, wb, biases):
    f32_spec = pl.BlockSpec((1, N_BLK, D), lambda b, n: (b, n, 0))
    return pl.pallas_call(
        _matmul_body,
        grid=(B, N // N_BLK),
        in_specs=[f32_spec] * 5 + [
            pl.BlockSpec((D, OUT), lambda b, n: (0, 0)),
            pl.BlockSpec((4, D, OUT), lambda b, n: (0, 0, 0)),
            pl.BlockSpec((OUT,), lambda b, n: (0,)),
        ],
        out_specs=pl.BlockSpec((1, N_BLK, OUT), lambda b, n: (b, n, 0)),
        out_shape=jax.ShapeDtypeStruct((B, N, OUT), jnp.float32),
    )(x0, s11, s21, s12, s22, w0, wb, biases)


def _pad_edges(src, dst, val):
    pad = EPAD - E
    fill = (jnp.arange(pad, dtype=jnp.int32) * 131) % N
    return (jnp.concatenate([src, fill]), jnp.concatenate([dst, fill]),
            jnp.pad(val, (0, pad)))


def kernel(x, edge_src1, edge_dst1, edge_val1, edge_src2, edge_dst2,
           edge_val2, weight, biases):
    b, n, d = x.shape
    s1, d1, v1 = _pad_edges(edge_src1, edge_dst1, edge_val1)
    s2, d2, v2 = _pad_edges(edge_src2, edge_dst2, edge_val2)

    xb = x.astype(jnp.bfloat16)
    s11 = _spmm_sc(xb, s1, d1, v1)
    s21 = _spmm_sc(s11, s1, d1, v1)
    s12 = _spmm_sc(xb, s2, d2, v2)
    s22 = _spmm_sc(s12, s2, d2, v2)

    w = weight.reshape(d, NUM_MAT, OUT)
    w0 = w[:, 0] - w[:, 2] - w[:, 4]
    wb = jnp.stack([
        w[:, 1],
        2.0 * w[:, 2],
        w[:, 3],
        2.0 * w[:, 4],
    ]).astype(jnp.bfloat16)
    return _final_matmul(x, s11, s21, s12, s22, w0, wb, biases)


# speedup vs baseline: 1.9858x; 1.0881x over previous
import jax
import jax.numpy as jnp
from jax import lax
from jax.experimental import pallas as pl
from jax.experimental.pallas import tpu as pltpu
from jax.experimental.pallas import tpu_sc as plsc

N = 10000
D = 128
B = 8
OUT = 128
NUM_MAT = 5
N_BLK = 2000

NC = 2
NS = 16
W = 128
CW = 16
CHE = CW * W
NCH = 10
NWIN = NCH * CW
EPS = NWIN * W
EPAD = EPS * NS
E = 320000
CHUNKS_PER_SC = B // NC
RS = 624
R_REM = N - NS * RS
NZ = 640

_GATHER_DNUMS = lax.GatherDimensionNumbers(
    offset_dims=(), collapsed_slice_dims=(0,), start_index_map=(0,))


def _bcast_lane(vec16, e):
    idx = jnp.full((16, 1), e, jnp.int32)
    return lax.gather(vec16, idx, _GATHER_DNUMS, slice_sizes=(1,),
                      mode=lax.GatherScatterMode.PROMISE_IN_BOUNDS)


def _spmm_body(x_hbm, src_hbm, dst_hbm, val_hbm, zeros_hbm, out_hbm,
               acc_sh, src_c, dst_c, val_c,
               r0, r1, r2, r3, sc0, sc1, dw0, dw1,
               g0, g1, g2, g3, ss0, ss1):
    cid = lax.axis_index("c")
    sid = lax.axis_index("s")
    ebase = sid * EPS
    rows = (r0, r1, r2, r3)
    gsem = (g0, g1, g2, g3)
    scaled = (sc0, sc1)
    ssem = (ss0, ss1)
    dw = (dw0, dw1)

    def emit_chunk(t, row_off):
        cbase = ebase + t * CHE
        pltpu.sync_copy(src_hbm.at[pl.ds(cbase, CHE)], src_c)
        pltpu.sync_copy(dst_hbm.at[pl.ds(cbase, CHE)], dst_c)
        pltpu.sync_copy(val_hbm.at[pl.ds(cbase, CHE)], val_c)

        @pl.loop(0, CHE // 16)
        def _(g):
            sl = pl.ds(g * 16, 16)
            src_c[sl] = src_c[sl] + row_off

        gd = [None] * CW
        for i in range(4):
            gd[i] = pltpu.async_copy(
                x_hbm.at[src_c.at[pl.ds(i * W, W)]], rows[i], gsem[i])

        for w in range(CW):
            p3, p2 = w % 4, w % 2
            gd[w].wait()

            def drain(p=p2):
                pltpu.make_async_copy(
                    scaled[p], acc_sh.at[dw[p]], ssem[p]).wait()

            if w >= 2:
                drain()
            else:
                pl.when(t > 0)(drain)

            @pl.loop(0, W // 16)
            def _(g):
                dw[p2][pl.ds(g * 16, 16)] = dst_c[pl.ds(w * W + g * 16, 16)]

            @pl.loop(0, W // 16)
            def _(g):
                v16 = val_c[pl.ds(w * W + g * 16, 16)]
                for e in range(16):
                    bc = plsc.pack(*[_bcast_lane(v16, e)] * 2,
                                   format=plsc.PackFormat.INTERLEAVED)
                    r = g * 16 + e
                    for q in range(4):
                        sl = pl.ds(q * 32, 32)
                        scaled[p2][r, sl] = rows[p3][r, sl] * bc

            if w < CW - 4:
                gd[w + 4] = pltpu.async_copy(
                    x_hbm.at[src_c.at[pl.ds((w + 4) * W, W)]],
                    rows[p3], gsem[p3])

            pltpu.async_copy(scaled[p2], acc_sh.at[dw[p2]], ssem[p2],
                             add=True)

    @pl.loop(0, CHUNKS_PER_SC)
    def _(bi):
        bb = cid * CHUNKS_PER_SC + bi
        row_off = bb * N

        pltpu.sync_copy(zeros_hbm.at[pl.ds(0, RS)],
                        acc_sh.at[pl.ds(sid * RS, RS)])

        @pl.when(sid == 0)
        def _():
            pltpu.sync_copy(zeros_hbm.at[pl.ds(0, R_REM)],
                            acc_sh.at[pl.ds(NS * RS, R_REM)])
        plsc.subcore_barrier()

        @pl.loop(0, NCH)
        def _(t):
            emit_chunk(t, row_off)

        for p2 in range(2):
            pltpu.make_async_copy(
                scaled[p2], acc_sh.at[dw[p2]], ssem[p2]).wait()
        plsc.subcore_barrier()

        pltpu.sync_copy(acc_sh.at[pl.ds(sid * RS, RS)],
                        out_hbm.at[bb].at[pl.ds(sid * RS, RS)])

        @pl.when(sid == 0)
        def _():
            pltpu.sync_copy(acc_sh.at[pl.ds(NS * RS, R_REM)],
                            out_hbm.at[bb].at[pl.ds(NS * RS, R_REM)])
        plsc.subcore_barrier()


@jax.jit
def _spmm_sc(x_cm, src, dst, val):
    mesh = plsc.VectorSubcoreMesh(core_axis_name="c", subcore_axis_name="s")
    x_flat = x_cm.reshape(B * N, D)
    cp = pltpu.CompilerParams(needs_layout_passes=False,
                              use_tc_tiling_on_sc=False)
    kern = pl.kernel(
        _spmm_body,
        out_type=jax.ShapeDtypeStruct((B, N, D), jnp.bfloat16),
        mesh=mesh,
        compiler_params=cp,
        scratch_types=[
            pltpu.VMEM_SHARED((N, D), jnp.bfloat16),
            pltpu.VMEM((CHE,), jnp.int32),
            pltpu.VMEM((CHE,), jnp.int32),
            pltpu.VMEM((CHE,), jnp.float32),
            pltpu.VMEM((W, D), jnp.bfloat16),
            pltpu.VMEM((W, D), jnp.bfloat16),
            pltpu.VMEM((W, D), jnp.bfloat16),
            pltpu.VMEM((W, D), jnp.bfloat16),
            pltpu.VMEM((W, D), jnp.bfloat16),
            pltpu.VMEM((W, D), jnp.bfloat16),
            pltpu.VMEM((W,), jnp.int32),
            pltpu.VMEM((W,), jnp.int32),
            pltpu.SemaphoreType.DMA,
            pltpu.SemaphoreType.DMA,
            pltpu.SemaphoreType.DMA,
            pltpu.SemaphoreType.DMA,
            pltpu.SemaphoreType.DMA,
            pltpu.SemaphoreType.DMA,
        ],
    )
    zeros = jnp.zeros((NZ, D), jnp.bfloat16)
    return kern(x_flat, src, dst, val, zeros)


def _matmul_body(x0_ref, s11_ref, s21_ref, s12_ref, s22_ref, w0_ref,
                 wb_ref, b_ref, o_ref):
    acc = jnp.dot(x0_ref[0], w0_ref[...], preferred_element_type=jnp.float32)
    acc += jnp.dot(s11_ref[0], wb_ref[0], preferred_element_type=jnp.float32)
    acc += jnp.dot(s21_ref[0], wb_ref[1], preferred_element_type=jnp.float32)
    acc += jnp.dot(s12_ref[0], wb_ref[2], preferred_element_type=jnp.float32)
    acc += jnp.dot(s22_ref[0], wb_ref[3], preferred_element_type=jnp.float32)
    o_ref[0] = acc + b_ref[...]


def _final_matmul(x0, s11, s21, s12, s22, w0, wb, biases):
    f32_spec = pl.BlockSpec((1, N_BLK, D), lambda b, n: (b, n, 0))
    return pl.pallas_call(
        _matmul_body,
        grid=(B, N // N_BLK),
        in_specs=[f32_spec] * 5 + [
            pl.BlockSpec((D, OUT), lambda b, n: (0, 0)),
            pl.BlockSpec((4, D, OUT), lambda b, n: (0, 0, 0)),
            pl.BlockSpec((OUT,), lambda b, n: (0,)),
        ],
        out_specs=pl.BlockSpec((1, N_BLK, OUT), lambda b, n: (b, n, 0)),
        out_shape=jax.ShapeDtypeStruct((B, N, OUT), jnp.float32),
    )(x0, s11, s21, s12, s22, w0, wb, biases)


def _pad_edges(src, dst, val):
    pad = EPAD - E
    fill = (jnp.arange(pad, dtype=jnp.int32) * 131) % N
    return (jnp.concatenate([src, fill]), jnp.concatenate([dst, fill]),
            jnp.pad(val, (0, pad)))


def kernel(x, edge_src1, edge_dst1, edge_val1, edge_src2, edge_dst2,
           edge_val2, weight, biases):
    b, n, d = x.shape
    s1, d1, v1 = _pad_edges(edge_src1, edge_dst1, edge_val1)
    s2, d2, v2 = _pad_edges(edge_src2, edge_dst2, edge_val2)

    xb = x.astype(jnp.bfloat16)
    s11 = _spmm_sc(xb, s1, d1, v1)
    s21 = _spmm_sc(s11, s1, d1, v1)
    s12 = _spmm_sc(xb, s2, d2, v2)
    s22 = _spmm_sc(s12, s2, d2, v2)

    w = weight.reshape(d, NUM_MAT, OUT)
    w0 = w[:, 0] - w[:, 2] - w[:, 4]
    wb = jnp.stack([
        w[:, 1],
        2.0 * w[:, 2],
        w[:, 3],
        2.0 * w[:, 4],
    ]).astype(jnp.bfloat16)
    return _final_matmul(x, s11, s21, s12, s22, w0, wb, biases)


# speedup vs baseline: 2.0410x; 1.0278x over previous
import jax
import jax.numpy as jnp
from jax import lax
from jax.experimental import pallas as pl
from jax.experimental.pallas import tpu as pltpu
from jax.experimental.pallas import tpu_sc as plsc

N = 10000
D = 128
B = 8
OUT = 128
NUM_MAT = 5
N_BLK = 2000

NC = 2
NS = 16
W = 128
CW = 20
CHE = CW * W
NCH = 8
NWIN = NCH * CW
EPS = NWIN * W
EPAD = EPS * NS
E = 320000
CHUNKS_PER_SC = B // NC
RS = 624
R_REM = N - NS * RS
NZ = 640

_GATHER_DNUMS = lax.GatherDimensionNumbers(
    offset_dims=(), collapsed_slice_dims=(0,), start_index_map=(0,))


def _bcast_lane(vec16, e):
    idx = jnp.full((16, 1), e, jnp.int32)
    return lax.gather(vec16, idx, _GATHER_DNUMS, slice_sizes=(1,),
                      mode=lax.GatherScatterMode.PROMISE_IN_BOUNDS)


def _spmm_body(x_hbm, src_hbm, dst_hbm, val_hbm, zeros_hbm, out_hbm,
               acc_sh, src_c, dst_c, val_c,
               r0, r1, r2, r3, sc0, sc1, dw0, dw1,
               g0, g1, g2, g3, ss0, ss1):
    cid = lax.axis_index("c")
    sid = lax.axis_index("s")
    ebase = sid * EPS
    rows = (r0, r1, r2, r3)
    gsem = (g0, g1, g2, g3)
    scaled = (sc0, sc1)
    ssem = (ss0, ss1)
    dw = (dw0, dw1)

    def emit_chunk(t, row_off):
        cbase = ebase + t * CHE
        pltpu.sync_copy(src_hbm.at[pl.ds(cbase, CHE)], src_c)
        pltpu.sync_copy(dst_hbm.at[pl.ds(cbase, CHE)], dst_c)
        pltpu.sync_copy(val_hbm.at[pl.ds(cbase, CHE)], val_c)

        @pl.loop(0, CHE // 16)
        def _(g):
            sl = pl.ds(g * 16, 16)
            src_c[sl] = src_c[sl] + row_off

        gd = [None] * CW
        for i in range(4):
            gd[i] = pltpu.async_copy(
                x_hbm.at[src_c.at[pl.ds(i * W, W)]], rows[i], gsem[i])

        for w in range(CW):
            p3, p2 = w % 4, w % 2
            gd[w].wait()

            def drain(p=p2):
                pltpu.make_async_copy(
                    scaled[p], acc_sh.at[dw[p]], ssem[p]).wait()

            if w >= 2:
                drain()
            else:
                pl.when(t > 0)(drain)

            @pl.loop(0, W // 16)
            def _(g):
                dw[p2][pl.ds(g * 16, 16)] = dst_c[pl.ds(w * W + g * 16, 16)]

            @pl.loop(0, W // 16)
            def _(g):
                v16 = val_c[pl.ds(w * W + g * 16, 16)]
                for e in range(16):
                    bc = plsc.pack(*[_bcast_lane(v16, e)] * 2,
                                   format=plsc.PackFormat.INTERLEAVED)
                    r = g * 16 + e
                    for q in range(4):
                        sl = pl.ds(q * 32, 32)
                        scaled[p2][r, sl] = rows[p3][r, sl] * bc

            if w < CW - 4:
                gd[w + 4] = pltpu.async_copy(
                    x_hbm.at[src_c.at[pl.ds((w + 4) * W, W)]],
                    rows[p3], gsem[p3])

            pltpu.async_copy(scaled[p2], acc_sh.at[dw[p2]], ssem[p2],
                             add=True)

    @pl.loop(0, CHUNKS_PER_SC)
    def _(bi):
        bb = cid * CHUNKS_PER_SC + bi
        row_off = bb * N

        pltpu.sync_copy(zeros_hbm.at[pl.ds(0, RS)],
                        acc_sh.at[pl.ds(sid * RS, RS)])

        @pl.when(sid == 0)
        def _():
            pltpu.sync_copy(zeros_hbm.at[pl.ds(0, R_REM)],
                            acc_sh.at[pl.ds(NS * RS, R_REM)])
        plsc.subcore_barrier()

        @pl.loop(0, NCH)
        def _(t):
            emit_chunk(t, row_off)

        for p2 in range(2):
            pltpu.make_async_copy(
                scaled[p2], acc_sh.at[dw[p2]], ssem[p2]).wait()
        plsc.subcore_barrier()

        pltpu.sync_copy(acc_sh.at[pl.ds(sid * RS, RS)],
                        out_hbm.at[bb].at[pl.ds(sid * RS, RS)])

        @pl.when(sid == 0)
        def _():
            pltpu.sync_copy(acc_sh.at[pl.ds(NS * RS, R_REM)],
                            out_hbm.at[bb].at[pl.ds(NS * RS, R_REM)])
        plsc.subcore_barrier()


@jax.jit
def _spmm_sc(x_cm, src, dst, val):
    mesh = plsc.VectorSubcoreMesh(core_axis_name="c", subcore_axis_name="s")
    x_flat = x_cm.reshape(B * N, D)
    cp = pltpu.CompilerParams(needs_layout_passes=False,
                              use_tc_tiling_on_sc=False)
    kern = pl.kernel(
        _spmm_body,
        out_type=jax.ShapeDtypeStruct((B, N, D), jnp.bfloat16),
        mesh=mesh,
        compiler_params=cp,
        scratch_types=[
            pltpu.VMEM_SHARED((N, D), jnp.bfloat16),
            pltpu.VMEM((CHE,), jnp.int32),
            pltpu.VMEM((CHE,), jnp.int32),
            pltpu.VMEM((CHE,), jnp.float32),
            pltpu.VMEM((W, D), jnp.bfloat16),
            pltpu.VMEM((W, D), jnp.bfloat16),
            pltpu.VMEM((W, D), jnp.bfloat16),
            pltpu.VMEM((W, D), jnp.bfloat16),
            pltpu.VMEM((W, D), jnp.bfloat16),
            pltpu.VMEM((W, D), jnp.bfloat16),
            pltpu.VMEM((W,), jnp.int32),
            pltpu.VMEM((W,), jnp.int32),
            pltpu.SemaphoreType.DMA,
            pltpu.SemaphoreType.DMA,
            pltpu.SemaphoreType.DMA,
            pltpu.SemaphoreType.DMA,
            pltpu.SemaphoreType.DMA,
            pltpu.SemaphoreType.DMA,
        ],
    )
    zeros = jnp.zeros((NZ, D), jnp.bfloat16)
    return kern(x_flat, src, dst, val, zeros)


def _matmul_body(x0_ref, s11_ref, s21_ref, s12_ref, s22_ref, w0_ref,
                 wb_ref, b_ref, o_ref):
    acc = jnp.dot(x0_ref[0], w0_ref[...], preferred_element_type=jnp.float32)
    acc += jnp.dot(s11_ref[0], wb_ref[0], preferred_element_type=jnp.float32)
    acc += jnp.dot(s21_ref[0], wb_ref[1], preferred_element_type=jnp.float32)
    acc += jnp.dot(s12_ref[0], wb_ref[2], preferred_element_type=jnp.float32)
    acc += jnp.dot(s22_ref[0], wb_ref[3], preferred_element_type=jnp.float32)
    o_ref[0] = acc + b_ref[...]


def _final_matmul(x0, s11, s21, s12, s22, w0, wb, biases):
    f32_spec = pl.BlockSpec((1, N_BLK, D), lambda b, n: (b, n, 0))
    return pl.pallas_call(
        _matmul_body,
        grid=(B, N // N_BLK),
        in_specs=[f32_spec] * 5 + [
            pl.BlockSpec((D, OUT), lambda b, n: (0, 0)),
            pl.BlockSpec((4, D, OUT), lambda b, n: (0, 0, 0)),
            pl.BlockSpec((OUT,), lambda b, n: (0,)),
        ],
        out_specs=pl.BlockSpec((1, N_BLK, OUT), lambda b, n: (b, n, 0)),
        out_shape=jax.ShapeDtypeStruct((B, N, OUT), jnp.float32),
    )(x0, s11, s21, s12, s22, w0, wb, biases)


def _pad_edges(src, dst, val):
    pad = EPAD - E
    fill = (jnp.arange(pad, dtype=jnp.int32) * 131) % N
    return (jnp.concatenate([src, fill]), jnp.concatenate([dst, fill]),
            jnp.pad(val, (0, pad)))


def kernel(x, edge_src1, edge_dst1, edge_val1, edge_src2, edge_dst2,
           edge_val2, weight, biases):
    b, n, d = x.shape
    s1, d1, v1 = _pad_edges(edge_src1, edge_dst1, edge_val1)
    s2, d2, v2 = _pad_edges(edge_src2, edge_dst2, edge_val2)

    xb = x.astype(jnp.bfloat16)
    s11 = _spmm_sc(xb, s1, d1, v1)
    s21 = _spmm_sc(s11, s1, d1, v1)
    s12 = _spmm_sc(xb, s2, d2, v2)
    s22 = _spmm_sc(s12, s2, d2, v2)

    w = weight.reshape(d, NUM_MAT, OUT)
    w0 = w[:, 0] - w[:, 2] - w[:, 4]
    wb = jnp.stack([
        w[:, 1],
        2.0 * w[:, 2],
        w[:, 3],
        2.0 * w[:, 4],
    ]).astype(jnp.bfloat16)
    return _final_matmul(x, s11, s21, s12, s22, w0, wb, biases)
